# speedup vs baseline: 2.0069x; 2.0069x over previous
import functools

import jax
import jax.numpy as jnp
from jax import lax
from jax.experimental import pallas as pl
from jax.experimental.pallas import tpu as pltpu
from jax.experimental.pallas import tpu_sc as plsc

_D_MODEL = 256
_BATCH = 16384

_info = plsc.get_sparse_core_info()
_NC, _NS = _info.num_cores, _info.num_subcores
_NW = _NC * _NS
_BPW = _BATCH // _NW
_CHUNK = 128
_NCHUNK = _BPW // _CHUNK

_mesh = plsc.VectorSubcoreMesh(core_axis_name="c", subcore_axis_name="s")


@functools.partial(
    pl.kernel,
    mesh=_mesh,
    out_type=jax.ShapeDtypeStruct((_BATCH, _D_MODEL), jnp.float32),
    scratch_types=[
        pltpu.VMEM((_NCHUNK, _CHUNK), jnp.int32),
        pltpu.VMEM((2, _CHUNK, _D_MODEL), jnp.float32),
        pltpu.SemaphoreType.DMA,
        pltpu.SemaphoreType.DMA,
    ],
)
def _gather_kernel(ts_hbm, emb_hbm, out_hbm, idx_v, rows_v, sem0, sem1):
    wid = lax.axis_index("s") * _NC + lax.axis_index("c")
    base = wid * _BPW
    for j in range(_NCHUNK):
        pltpu.sync_copy(ts_hbm.at[pl.ds(base + j * _CHUNK, _CHUNK)], idx_v.at[j])
    sems = (sem0, sem1)
    prev = pltpu.async_copy(emb_hbm.at[idx_v.at[0]], rows_v.at[0], sems[0])
    for j in range(1, _NCHUNK):
        cur = pltpu.async_copy(emb_hbm.at[idx_v.at[j]], rows_v.at[j % 2], sems[j % 2])
        prev.wait()
        pltpu.sync_copy(
            rows_v.at[(j - 1) % 2],
            out_hbm.at[pl.ds(base + (j - 1) * _CHUNK, _CHUNK)],
        )
        prev = cur
    prev.wait()
    pltpu.sync_copy(
        rows_v.at[(_NCHUNK - 1) % 2],
        out_hbm.at[pl.ds(base + (_NCHUNK - 1) * _CHUNK, _CHUNK)],
    )


def kernel(timesteps, embeddings):
    return _gather_kernel(timesteps.astype(jnp.int32), embeddings)


# speedup vs baseline: 2.0769x; 1.0349x over previous
import functools

import jax
import jax.numpy as jnp
from jax import lax
from jax.experimental import pallas as pl
from jax.experimental.pallas import tpu as pltpu
from jax.experimental.pallas import tpu_sc as plsc

_D_MODEL = 256
_BATCH = 16384

_info = plsc.get_sparse_core_info()
_NC, _NS = _info.num_cores, _info.num_subcores
_NW = _NC * _NS
_BPW = _BATCH // _NW
_CHUNK = 128
_NCHUNK = _BPW // _CHUNK

_mesh = plsc.VectorSubcoreMesh(core_axis_name="c", subcore_axis_name="s")


_NBUF = 3


@functools.partial(
    pl.kernel,
    mesh=_mesh,
    out_type=jax.ShapeDtypeStruct((_BATCH, _D_MODEL), jnp.float32),
    scratch_types=[
        pltpu.VMEM((_NCHUNK, _CHUNK), jnp.int32),
        pltpu.VMEM((_NBUF, _CHUNK, _D_MODEL), jnp.float32),
        [pltpu.SemaphoreType.DMA] * _NBUF,
        [pltpu.SemaphoreType.DMA] * _NBUF,
    ],
)
def _gather_kernel(ts_hbm, emb_hbm, out_hbm, idx_v, rows_v, gsems, wsems):
    wid = lax.axis_index("s") * _NC + lax.axis_index("c")
    base = wid * _BPW
    for j in range(_NCHUNK):
        pltpu.sync_copy(ts_hbm.at[pl.ds(base + j * _CHUNK, _CHUNK)], idx_v.at[j])
    gathers = [
        pltpu.async_copy(emb_hbm.at[idx_v.at[b]], rows_v.at[b], gsems[b])
        for b in range(_NBUF)
    ]
    wbs = [None] * _NBUF
    for j in range(_NCHUNK):
        b = j % _NBUF
        gathers[b].wait()
        wbs[b] = pltpu.async_copy(
            rows_v.at[b], out_hbm.at[pl.ds(base + j * _CHUNK, _CHUNK)], wsems[b]
        )
        nj = j + _NBUF
        if nj < _NCHUNK:
            wbs[b].wait()
            wbs[b] = None
            gathers[b] = pltpu.async_copy(
                emb_hbm.at[idx_v.at[nj]], rows_v.at[b], gsems[b]
            )
    for b in range(_NBUF):
        if wbs[b] is not None:
            wbs[b].wait()


def kernel(timesteps, embeddings):
    return _gather_kernel(timesteps.astype(jnp.int32), embeddings)


# speedup vs baseline: 2.1388x; 1.0298x over previous
import functools

import jax
import jax.numpy as jnp
from jax import lax
from jax.experimental import pallas as pl
from jax.experimental.pallas import tpu as pltpu
from jax.experimental.pallas import tpu_sc as plsc

_D_MODEL = 256
_BATCH = 16384

_info = plsc.get_sparse_core_info()
_NC, _NS = _info.num_cores, _info.num_subcores
_NW = _NC * _NS
_BPW = _BATCH // _NW
_CHUNK = 128
_NCHUNK = _BPW // _CHUNK
_NBUF = 3

_mesh = plsc.VectorSubcoreMesh(core_axis_name="c", subcore_axis_name="s")


@functools.partial(
    pl.kernel,
    mesh=_mesh,
    out_type=jax.ShapeDtypeStruct((_BATCH, _D_MODEL), jnp.float32),
    scratch_types=[
        pltpu.VMEM((_BPW,), jnp.int32),
        pltpu.VMEM((_NBUF, _CHUNK, _D_MODEL), jnp.float32),
        [pltpu.SemaphoreType.DMA] * _NBUF,
        [pltpu.SemaphoreType.DMA] * _NBUF,
    ],
)
def _gather_kernel(ts_hbm, emb_hbm, out_hbm, idx_v, rows_v, gsems, wsems):
    wid = lax.axis_index("s") * _NC + lax.axis_index("c")
    base = wid * _BPW
    pltpu.sync_copy(ts_hbm.at[pl.ds(base, _BPW)], idx_v)

    def _idx(j):
        return idx_v.at[pl.ds(j * _CHUNK, _CHUNK)]

    gathers = [
        pltpu.async_copy(emb_hbm.at[_idx(b)], rows_v.at[b], gsems[b])
        for b in range(_NBUF)
    ]
    wbs = [None] * _NBUF
    for j in range(_NCHUNK):
        b = j % _NBUF
        gathers[b].wait()
        wbs[b] = pltpu.async_copy(
            rows_v.at[b], out_hbm.at[pl.ds(base + j * _CHUNK, _CHUNK)], wsems[b]
        )
        nj = j + _NBUF
        if nj < _NCHUNK:
            wbs[b].wait()
            wbs[b] = None
            gathers[b] = pltpu.async_copy(
                emb_hbm.at[_idx(nj)], rows_v.at[b], gsems[b]
            )
    for b in range(_NBUF):
        if wbs[b] is not None:
            wbs[b].wait()


def kernel(timesteps, embeddings):
    return _gather_kernel(timesteps.astype(jnp.int32), embeddings)
